# baseline (speedup 1.0000x reference)
import functools

import jax
import jax.numpy as jnp
from jax import lax
from jax.experimental import pallas as pl
from jax.experimental.pallas import tpu as pltpu
from jax.experimental.pallas import tpu_sc as plsc

_E = 64
_H = 768
_I = 1024
_T = 2048
_CH = 256
_TM = 128
_TP = _T + _E * 8


def _router_body(x_ref, gw_ref, pos_ref, sizes_ref, offs_ref, aux_ref):
    nch = _T // _CH
    eidx = lax.broadcasted_iota(jnp.int32, (_CH, _E), 1)
    lmat = (lax.broadcasted_iota(jnp.int32, (_CH, _CH), 0)
            > lax.broadcasted_iota(jnp.int32, (_CH, _CH), 1)).astype(jnp.float32)
    base = jnp.zeros((1, _E), jnp.float32)
    imps = jnp.zeros((1, _E), jnp.float32)
    idxs, ranks = [], []
    for c in range(nch):
        xc = x_ref[c * _CH:(c + 1) * _CH, :]
        logits = lax.dot_general(xc, gw_ref[...], (((1,), (1,)), ((), ())),
                                 preferred_element_type=jnp.float32)
        m = jnp.max(logits, axis=-1, keepdims=True)
        p = jnp.exp(logits - m)
        probs = p / jnp.sum(p, axis=-1, keepdims=True)
        imps = imps + jnp.sum(probs, axis=0, keepdims=True)
        pm = jnp.max(probs, axis=-1, keepdims=True)
        amax = jnp.min(jnp.where(probs >= pm, eidx, _E), axis=-1, keepdims=True)
        oh = (eidx == amax).astype(jnp.float32)
        cum = lax.dot_general(lmat, oh, (((1,), (0,)), ((), ())),
                              preferred_element_type=jnp.float32)
        rank = jnp.sum((cum + base) * oh, axis=-1, keepdims=True)
        idxs.append(amax)
        ranks.append(rank)
        base = base + jnp.sum(oh, axis=0, keepdims=True)
    counts = base
    tri = (lax.broadcasted_iota(jnp.int32, (_E, _E), 0)
           < lax.broadcasted_iota(jnp.int32, (_E, _E), 1)).astype(jnp.float32)
    pcounts = jnp.floor((counts + 7.0) * 0.125) * 8.0
    offs = lax.dot_general(pcounts, tri, (((1,), (0,)), ((), ())),
                           preferred_element_type=jnp.float32)
    imp = imps * (1.0 / _T)
    load = counts * (1.0 / _T)
    mean = jnp.sum(imp) / _E
    var = jnp.sum((imp - mean) ** 2) / (_E - 1)
    aux = _E * jnp.sum(imp * load) + var * _E
    aux_ref[...] = jnp.reshape(aux, (1, 1))
    sizes_ref[...] = counts.astype(jnp.int32)
    offs_ref[...] = offs.astype(jnp.int32)
    for c in range(nch):
        oh = (eidx == idxs[c]).astype(jnp.float32)
        posf = ranks[c] + jnp.sum(oh * offs, axis=-1, keepdims=True)
        pos_ref[c * _CH:(c + 1) * _CH, :] = posf.astype(jnp.int32)


def _router(xf, gate_w):
    return pl.pallas_call(
        _router_body,
        out_shape=[
            jax.ShapeDtypeStruct((_T, 1), jnp.int32),
            jax.ShapeDtypeStruct((1, _E), jnp.int32),
            jax.ShapeDtypeStruct((1, _E), jnp.int32),
            jax.ShapeDtypeStruct((1, 1), jnp.float32),
        ],
    )(xf, gate_w)


_BI = _I // 2


def _ffn_body(sizes_ref, offs_ref, xs_ref, wg_ref, wu_ref, wd_ref, ys_ref):
    e = pl.program_id(0)
    ki = pl.program_id(1)
    n = sizes_ref[0, e]
    off = offs_ref[0, e]
    nt = (n + _TM - 1) // _TM

    def body(j, carry):
        start = off + j * _TM
        start_c = pl.multiple_of(jnp.minimum(start, _TP - _TM), 8)
        xq = xs_ref[pl.ds(start_c, _TM), :]
        g = lax.dot_general(xq, wg_ref[0], (((1,), (1,)), ((), ())),
                            preferred_element_type=jnp.float32)
        u = lax.dot_general(xq, wu_ref[0], (((1,), (1,)), ((), ())),
                            preferred_element_type=jnp.float32)
        h = g * jax.nn.sigmoid(g) * u
        o = lax.dot_general(h, wd_ref[0], (((1,), (1,)), ((), ())),
                            preferred_element_type=jnp.float32)
        row = start_c + lax.broadcasted_iota(jnp.int32, (_TM, 1), 0)
        valid = (row >= start) & (row < off + n)
        cur = ys_ref[pl.ds(start_c, _TM), :]
        acc = jnp.where(ki == 0, o, o + cur)
        ys_ref[pl.ds(start_c, _TM), :] = jnp.where(valid, acc, cur)
        return carry

    lax.fori_loop(0, nt, body, 0)


def _ffn(sizes, offs, xs, w_gate, w_up, w_down):
    return pl.pallas_call(
        _ffn_body,
        grid=(_E, _I // _BI),
        in_specs=[
            pl.BlockSpec(memory_space=pltpu.SMEM),
            pl.BlockSpec(memory_space=pltpu.SMEM),
            pl.BlockSpec((_TP, _H), lambda e, ki: (0, 0)),
            pl.BlockSpec((1, _BI, _H), lambda e, ki: (e, ki, 0)),
            pl.BlockSpec((1, _BI, _H), lambda e, ki: (e, ki, 0)),
            pl.BlockSpec((1, _H, _BI), lambda e, ki: (e, 0, ki)),
        ],
        out_specs=pl.BlockSpec((_TP, _H), lambda e, ki: (0, 0)),
        out_shape=jax.ShapeDtypeStruct((_TP, _H), jnp.float32),
        compiler_params=pltpu.CompilerParams(
            dimension_semantics=("arbitrary", "arbitrary")),
    )(sizes, offs, xs, w_gate, w_up, w_down)


def _sc_dispatch(xf, pos):
    info = plsc.get_sparse_core_info()
    nc, ns = info.num_cores, info.num_subcores
    rpw = _T // (nc * ns)
    mesh = plsc.VectorSubcoreMesh(core_axis_name="c", subcore_axis_name="s")

    @functools.partial(
        pl.kernel, mesh=mesh,
        out_type=jax.ShapeDtypeStruct((_TP, _H), jnp.float32),
        scratch_types=[
            pltpu.VMEM((rpw,), jnp.int32),
            pltpu.VMEM((rpw, _H), jnp.float32),
            pltpu.SemaphoreType.DMA,
            pltpu.SemaphoreType.DMA,
        ],
    )
    def disp(x_hbm, pos_hbm, xs_hbm, idx_v, rows_v, sem, sem2):
        wid = lax.axis_index("s") * nc + lax.axis_index("c")
        b = wid * rpw
        c1 = pltpu.async_copy(pos_hbm.at[pl.ds(b, rpw)], idx_v, sem)
        c2 = pltpu.async_copy(x_hbm.at[pl.ds(b, rpw)], rows_v, sem2)
        c1.wait()
        c2.wait()
        pltpu.async_copy(rows_v, xs_hbm.at[idx_v], sem).wait()

    return disp(xf, pos)


def _sc_combine(ys, pos):
    info = plsc.get_sparse_core_info()
    nc, ns = info.num_cores, info.num_subcores
    rpw = _T // (nc * ns)
    mesh = plsc.VectorSubcoreMesh(core_axis_name="c", subcore_axis_name="s")

    @functools.partial(
        pl.kernel, mesh=mesh,
        out_type=jax.ShapeDtypeStruct((_T, _H), jnp.float32),
        scratch_types=[
            pltpu.VMEM((rpw,), jnp.int32),
            pltpu.VMEM((rpw, _H), jnp.float32),
            pltpu.SemaphoreType.DMA,
        ],
    )
    def comb(ys_hbm, pos_hbm, out_hbm, idx_v, rows_v, sem):
        wid = lax.axis_index("s") * nc + lax.axis_index("c")
        b = wid * rpw
        pltpu.sync_copy(pos_hbm.at[pl.ds(b, rpw)], idx_v)
        pltpu.async_copy(ys_hbm.at[idx_v], rows_v, sem).wait()
        pltpu.sync_copy(rows_v, out_hbm.at[pl.ds(b, rpw)])

    return comb(ys, pos)


def kernel(x, gate_w, w_gate, w_up, w_down):
    b, s, hd = x.shape
    xf = x.reshape(_T, _H)
    pos2d, sizes, offs, aux = _router(xf, gate_w)
    pos = pos2d.reshape(_T)
    xs = _sc_dispatch(xf, pos)
    ys = _ffn(sizes, offs, xs, w_gate, w_up, w_down)
    routed = _sc_combine(ys, pos)
    return routed.reshape(b, s, hd), aux.reshape(())


# speedup vs baseline: 1.2084x; 1.2084x over previous
import functools

import jax
import jax.numpy as jnp
from jax import lax
from jax.experimental import pallas as pl
from jax.experimental.pallas import tpu as pltpu
from jax.experimental.pallas import tpu_sc as plsc

_E = 64
_H = 768
_I = 1024
_T = 2048
_CH = 256
_TM = 128
_TP = _T + _E * 8


def _router_body(x_ref, gw_ref, pos_ref, sizes_ref, offs_ref, aux_ref):
    nch = _T // _CH
    eidx = lax.broadcasted_iota(jnp.int32, (_CH, _E), 1)
    lmat = (lax.broadcasted_iota(jnp.int32, (_CH, _CH), 0)
            > lax.broadcasted_iota(jnp.int32, (_CH, _CH), 1)).astype(jnp.float32)
    base = jnp.zeros((1, _E), jnp.float32)
    imps = jnp.zeros((1, _E), jnp.float32)
    idxs, ranks = [], []
    for c in range(nch):
        xc = x_ref[c * _CH:(c + 1) * _CH, :]
        logits = lax.dot_general(xc, gw_ref[...], (((1,), (1,)), ((), ())),
                                 preferred_element_type=jnp.float32)
        m = jnp.max(logits, axis=-1, keepdims=True)
        p = jnp.exp(logits - m)
        probs = p / jnp.sum(p, axis=-1, keepdims=True)
        imps = imps + jnp.sum(probs, axis=0, keepdims=True)
        pm = jnp.max(probs, axis=-1, keepdims=True)
        amax = jnp.min(jnp.where(probs >= pm, eidx, _E), axis=-1, keepdims=True)
        oh = (eidx == amax).astype(jnp.float32)
        cum = lax.dot_general(lmat, oh, (((1,), (0,)), ((), ())),
                              preferred_element_type=jnp.float32)
        rank = jnp.sum((cum + base) * oh, axis=-1, keepdims=True)
        idxs.append(amax)
        ranks.append(rank)
        base = base + jnp.sum(oh, axis=0, keepdims=True)
    counts = base
    tri = (lax.broadcasted_iota(jnp.int32, (_E, _E), 0)
           < lax.broadcasted_iota(jnp.int32, (_E, _E), 1)).astype(jnp.float32)
    pcounts = jnp.floor((counts + 7.0) * 0.125) * 8.0
    offs = lax.dot_general(pcounts, tri, (((1,), (0,)), ((), ())),
                           preferred_element_type=jnp.float32)
    imp = imps * (1.0 / _T)
    load = counts * (1.0 / _T)
    mean = jnp.sum(imp) / _E
    var = jnp.sum((imp - mean) ** 2) / (_E - 1)
    aux = _E * jnp.sum(imp * load) + var * _E
    aux_ref[...] = jnp.reshape(aux, (1, 1))
    sizes_ref[...] = counts.astype(jnp.int32)
    offs_ref[...] = offs.astype(jnp.int32)
    for c in range(nch):
        oh = (eidx == idxs[c]).astype(jnp.float32)
        posf = ranks[c] + jnp.sum(oh * offs, axis=-1, keepdims=True)
        pos_ref[c * _CH:(c + 1) * _CH, :] = posf.astype(jnp.int32)


def _router(xf, gate_w):
    return pl.pallas_call(
        _router_body,
        out_shape=[
            jax.ShapeDtypeStruct((_T, 1), jnp.int32),
            jax.ShapeDtypeStruct((1, _E), jnp.int32),
            jax.ShapeDtypeStruct((1, _E), jnp.int32),
            jax.ShapeDtypeStruct((1, 1), jnp.float32),
        ],
    )(xf, gate_w)


_EB = 2


def _ffn_body(sizes_ref, offs_ref, xs_ref, wg_ref, wu_ref, wd_ref, ys_ref):
    eg = pl.program_id(0)
    for i in range(_EB):
        e = eg * _EB + i
        n = sizes_ref[0, e]
        off = offs_ref[0, e]
        nt = (n + _TM - 1) // _TM

        def body(j, carry):
            start = off + j * _TM
            start_c = pl.multiple_of(jnp.minimum(start, _TP - _TM), 8)
            xq = xs_ref[pl.ds(start_c, _TM), :]
            g = lax.dot_general(xq, wg_ref[i], (((1,), (1,)), ((), ())),
                                preferred_element_type=jnp.float32)
            u = lax.dot_general(xq, wu_ref[i], (((1,), (1,)), ((), ())),
                                preferred_element_type=jnp.float32)
            h = g * jax.nn.sigmoid(g) * u
            o = lax.dot_general(h, wd_ref[i], (((1,), (1,)), ((), ())),
                                preferred_element_type=jnp.float32)
            row = start_c + lax.broadcasted_iota(jnp.int32, (_TM, 1), 0)
            valid = (row >= start) & (row < off + n)
            cur = ys_ref[pl.ds(start_c, _TM), :]
            ys_ref[pl.ds(start_c, _TM), :] = jnp.where(valid, o, cur)
            return carry

        lax.fori_loop(0, nt, body, 0)


def _ffn(sizes, offs, xs, w_gate, w_up, w_down):
    return pl.pallas_call(
        _ffn_body,
        grid=(_E // _EB,),
        in_specs=[
            pl.BlockSpec(memory_space=pltpu.SMEM),
            pl.BlockSpec(memory_space=pltpu.SMEM),
            pl.BlockSpec((_TP, _H), lambda e: (0, 0)),
            pl.BlockSpec((_EB, _I, _H), lambda e: (e, 0, 0)),
            pl.BlockSpec((_EB, _I, _H), lambda e: (e, 0, 0)),
            pl.BlockSpec((_EB, _H, _I), lambda e: (e, 0, 0)),
        ],
        out_specs=pl.BlockSpec((_TP, _H), lambda e: (0, 0)),
        out_shape=jax.ShapeDtypeStruct((_TP, _H), jnp.float32),
        compiler_params=pltpu.CompilerParams(
            dimension_semantics=("arbitrary",)),
    )(sizes, offs, xs, w_gate, w_up, w_down)


def _sc_dispatch(xf, pos):
    info = plsc.get_sparse_core_info()
    nc, ns = info.num_cores, info.num_subcores
    rpw = _T // (nc * ns)
    mesh = plsc.VectorSubcoreMesh(core_axis_name="c", subcore_axis_name="s")

    @functools.partial(
        pl.kernel, mesh=mesh,
        out_type=jax.ShapeDtypeStruct((_TP, _H), jnp.float32),
        scratch_types=[
            pltpu.VMEM((rpw,), jnp.int32),
            pltpu.VMEM((rpw, _H), jnp.float32),
            pltpu.SemaphoreType.DMA,
            pltpu.SemaphoreType.DMA,
        ],
    )
    def disp(x_hbm, pos_hbm, xs_hbm, idx_v, rows_v, sem, sem2):
        wid = lax.axis_index("s") * nc + lax.axis_index("c")
        b = wid * rpw
        c1 = pltpu.async_copy(pos_hbm.at[pl.ds(b, rpw)], idx_v, sem)
        c2 = pltpu.async_copy(x_hbm.at[pl.ds(b, rpw)], rows_v, sem2)
        c1.wait()
        c2.wait()
        pltpu.async_copy(rows_v, xs_hbm.at[idx_v], sem).wait()

    return disp(xf, pos)


def _sc_combine(ys, pos):
    info = plsc.get_sparse_core_info()
    nc, ns = info.num_cores, info.num_subcores
    rpw = _T // (nc * ns)
    mesh = plsc.VectorSubcoreMesh(core_axis_name="c", subcore_axis_name="s")

    @functools.partial(
        pl.kernel, mesh=mesh,
        out_type=jax.ShapeDtypeStruct((_T, _H), jnp.float32),
        scratch_types=[
            pltpu.VMEM((rpw,), jnp.int32),
            pltpu.VMEM((rpw, _H), jnp.float32),
            pltpu.SemaphoreType.DMA,
        ],
    )
    def comb(ys_hbm, pos_hbm, out_hbm, idx_v, rows_v, sem):
        wid = lax.axis_index("s") * nc + lax.axis_index("c")
        b = wid * rpw
        pltpu.sync_copy(pos_hbm.at[pl.ds(b, rpw)], idx_v)
        pltpu.async_copy(ys_hbm.at[idx_v], rows_v, sem).wait()
        pltpu.sync_copy(rows_v, out_hbm.at[pl.ds(b, rpw)])

    return comb(ys, pos)


def kernel(x, gate_w, w_gate, w_up, w_down):
    b, s, hd = x.shape
    xf = x.reshape(_T, _H)
    pos2d, sizes, offs, aux = _router(xf, gate_w)
    pos = pos2d.reshape(_T)
    xs = _sc_dispatch(xf, pos)
    ys = _ffn(sizes, offs, xs, w_gate, w_up, w_down)
    routed = _sc_combine(ys, pos)
    return routed.reshape(b, s, hd), aux.reshape(())
